# Initial kernel scaffold; baseline (speedup 1.0000x reference)
import jax
import jax.numpy as jnp
from jax.experimental import pallas as pl


def kernel(x_patient, x_medicine, edge_index_pm, edge_index_mp, edge_label_index, W1l_pm, W1r_pm, b1_pm, W1l_mp, W1r_mp, b1_mp, W2l_pm, W2r_pm, b2_pm, W2l_mp, W2r_mp, b2_mp, Wd1, bd1, Wd2, bd2):
    raise NotImplementedError("write your pallas kernel here")


# speedup vs baseline: 2.7358x; 2.7358x over previous
import functools

import jax
import jax.numpy as jnp
from jax import lax
from jax.experimental import pallas as pl
from jax.experimental.pallas import tpu as pltpu
from jax.experimental.pallas import tpu_sc as plsc

N_NODE = 10000
E_EDGE = 160000
L_LAB = 20000
D = 256
HALF = 128

_CH = 80
_NCHUNK = E_EDGE // _CH
_ZROWS = 80
_NROWCH = N_NODE // _ZROWS

f32 = jnp.float32


@functools.lru_cache(maxsize=None)
def _get_mesh():
    return plsc.VectorSubcoreMesh(core_axis_name="c", subcore_axis_name="s")


@functools.lru_cache(maxsize=None)
def _make_segsum():
    @functools.partial(
        pl.kernel,
        out_type=[jax.ShapeDtypeStruct((2, N_NODE, HALF), f32)],
        mesh=_get_mesh(),
        scratch_types=[
            pltpu.VMEM((_CH,), jnp.int32),
            pltpu.VMEM((_CH,), jnp.int32),
            pltpu.VMEM((_CH,), jnp.int32),
            pltpu.VMEM((_CH, HALF), f32),
            pltpu.VMEM((_ZROWS, HALF), f32),
            pltpu.VMEM_SHARED((N_NODE, HALF), f32),
            pltpu.SemaphoreType.DMA,
        ],
    )
    def segsum(xs, src_hbm, dst_hbm, zblk, sums_hbm,
               src_v, idx_v, dst_v, rows_v, zrow_v, acc_sh, sem):
        c = lax.axis_index("c")
        s = lax.axis_index("s")
        pltpu.sync_copy(zblk, zrow_v)
        for j in range(8):
            ch = s + j * 16

            @pl.when(ch < _NROWCH)
            def _():
                pltpu.sync_copy(zrow_v, acc_sh.at[pl.ds(ch * _ZROWS, _ZROWS)])
        plsc.subcore_barrier()

        def step(i, carry):
            base = (i * 16 + s) * _CH
            pltpu.sync_copy(src_hbm.at[pl.ds(base, _CH)], src_v)
            pltpu.sync_copy(dst_hbm.at[pl.ds(base, _CH)], dst_v)
            for k in range(_CH // 16):
                sl = pl.ds(k * 16, 16)
                idx_v[sl] = src_v[sl] * 2 + c
            pltpu.async_copy(xs.at[idx_v], rows_v, sem).wait()
            pltpu.sync_copy(rows_v, acc_sh.at[dst_v], add=True)
            return carry

        lax.fori_loop(0, _NCHUNK // 16, step, 0)
        plsc.subcore_barrier()
        for j in range(8):
            ch = s + j * 16

            @pl.when(ch < _NROWCH)
            def _():
                r0 = ch * _ZROWS
                pltpu.sync_copy(acc_sh.at[pl.ds(r0, _ZROWS)],
                                sums_hbm.at[c, pl.ds(r0, _ZROWS)])

    return segsum


@functools.lru_cache(maxsize=None)
def _make_counts():
    @functools.partial(
        pl.kernel,
        out_type=[jax.ShapeDtypeStruct((2, N_NODE, HALF), f32)],
        mesh=_get_mesh(),
        scratch_types=[
            pltpu.VMEM((_CH,), jnp.int32),
            pltpu.VMEM((_CH, HALF), f32),
            pltpu.VMEM((_ZROWS, HALF), f32),
            pltpu.VMEM_SHARED((N_NODE, HALF), f32),
        ],
    )
    def counts(dst2_hbm, zblk, oneblk, cnts_hbm, dst_v, ones_v, zrow_v,
               cnt_sh):
        c = lax.axis_index("c")
        s = lax.axis_index("s")
        pltpu.sync_copy(zblk, zrow_v)
        pltpu.sync_copy(oneblk, ones_v)
        for j in range(8):
            ch = s + j * 16

            @pl.when(ch < _NROWCH)
            def _():
                pltpu.sync_copy(zrow_v, cnt_sh.at[pl.ds(ch * _ZROWS, _ZROWS)])
        plsc.subcore_barrier()

        def step(i, carry):
            base = c * E_EDGE + (i * 16 + s) * _CH
            pltpu.sync_copy(dst2_hbm.at[pl.ds(base, _CH)], dst_v)
            pltpu.sync_copy(ones_v, cnt_sh.at[dst_v], add=True)
            return carry

        lax.fori_loop(0, _NCHUNK // 16, step, 0)
        plsc.subcore_barrier()
        for j in range(8):
            ch = s + j * 16

            @pl.when(ch < _NROWCH)
            def _():
                r0 = ch * _ZROWS
                pltpu.sync_copy(cnt_sh.at[pl.ds(r0, _ZROWS)],
                                cnts_hbm.at[c, pl.ds(r0, _ZROWS)])

    return counts


_GCH = 80
_NGCH = L_LAB // _GCH


@functools.lru_cache(maxsize=None)
def _make_edge_gather():
    @functools.partial(
        pl.kernel,
        out_type=[jax.ShapeDtypeStruct((L_LAB, D), f32),
                  jax.ShapeDtypeStruct((L_LAB, D), f32)],
        mesh=_get_mesh(),
        scratch_types=[
            pltpu.VMEM((_GCH,), jnp.int32),
            pltpu.VMEM((_GCH,), jnp.int32),
            pltpu.VMEM((_GCH, D), f32),
            pltpu.VMEM((_GCH, D), f32),
            pltpu.SemaphoreType.DMA,
        ],
    )
    def edge_gather(p1, m1, row_hbm, col_hbm, pg_hbm, mg_hbm,
                    idx_v, idx2_v, rows_v, rows2_v, sem):
        c = lax.axis_index("c")
        s = lax.axis_index("s")
        wid = s * 2 + c

        def step(i, carry):
            ch = wid + i * 32

            @pl.when(ch < _NGCH)
            def _():
                base = ch * _GCH
                pltpu.sync_copy(row_hbm.at[pl.ds(base, _GCH)], idx_v)
                pltpu.async_copy(p1.at[idx_v], rows_v, sem).wait()
                pltpu.sync_copy(rows_v, pg_hbm.at[pl.ds(base, _GCH)])
                pltpu.sync_copy(col_hbm.at[pl.ds(base, _GCH)], idx2_v)
                pltpu.async_copy(m1.at[idx2_v], rows2_v, sem).wait()
                pltpu.sync_copy(rows2_v, mg_hbm.at[pl.ds(base, _GCH)])
            return carry

        lax.fori_loop(0, (_NGCH + 31) // 32, step, 0)

    return edge_gather


_BM = 1000


def _make_fused(relu: bool, with_post: bool):
    def body(*refs):
        if with_post:
            (sa_ref, sb_ref, cnt_ref, x_ref, wl_ref, wr_ref, b_ref, wp_ref,
             out_ref) = refs
        else:
            (sa_ref, sb_ref, cnt_ref, x_ref, wl_ref, wr_ref, b_ref,
             out_ref) = refs
        inv = 1.0 / jnp.maximum(cnt_ref[...], 1.0)
        wl = wl_ref[...]
        wr = wr_ref[...]
        b = b_ref[...]
        if with_post:
            wp = wp_ref[...]
            wl = jnp.dot(wl, wp, preferred_element_type=f32)
            wr = jnp.dot(wr, wp, preferred_element_type=f32)
            b = jnp.dot(b, wp, preferred_element_type=f32)
        acc = (jnp.dot(sa_ref[0] * inv, wl[:HALF, :],
                       preferred_element_type=f32)
               + jnp.dot(sb_ref[0] * inv, wl[HALF:, :],
                         preferred_element_type=f32)
               + jnp.dot(x_ref[...], wr, preferred_element_type=f32) + b)
        if relu:
            acc = jnp.maximum(acc, 0.0)
        out_ref[...] = acc

    n_rows = N_NODE // _BM
    in_specs = [
        pl.BlockSpec((1, _BM, HALF), lambda i: (0, i, 0)),
        pl.BlockSpec((1, _BM, HALF), lambda i: (1, i, 0)),
        pl.BlockSpec((_BM, 1), lambda i: (i, 0)),
        pl.BlockSpec((_BM, D), lambda i: (i, 0)),
        pl.BlockSpec((D, D), lambda i: (0, 0)),
        pl.BlockSpec((D, D), lambda i: (0, 0)),
        pl.BlockSpec((1, D), lambda i: (0, 0)),
    ]
    if with_post:
        in_specs.append(pl.BlockSpec((D, D), lambda i: (0, 0)))
    return pl.pallas_call(
        body,
        grid=(n_rows,),
        in_specs=in_specs,
        out_specs=pl.BlockSpec((_BM, D), lambda i: (i, 0)),
        out_shape=jax.ShapeDtypeStruct((N_NODE, D), f32),
    )


_fused_relu = _make_fused(True, False)
_fused_post = _make_fused(False, True)

_BL = 2000


def _dec_body(pg_ref, mg_ref, bd1_ref, wd2_ref, bd2_ref, out_ref):
    z = jnp.maximum(pg_ref[...] + mg_ref[...] + bd1_ref[...], 0.0)
    out_ref[...] = jnp.dot(z, wd2_ref[...], preferred_element_type=f32) + bd2_ref[...]


_decoder = pl.pallas_call(
    _dec_body,
    grid=(L_LAB // _BL,),
    in_specs=[
        pl.BlockSpec((_BL, D), lambda i: (i, 0)),
        pl.BlockSpec((_BL, D), lambda i: (i, 0)),
        pl.BlockSpec((1, D), lambda i: (0, 0)),
        pl.BlockSpec((D, 1), lambda i: (0, 0)),
        pl.BlockSpec((1, 1), lambda i: (0, 0)),
    ],
    out_specs=pl.BlockSpec((_BL, 1), lambda i: (i, 0)),
    out_shape=jax.ShapeDtypeStruct((L_LAB, 1), f32),
)


def kernel(x_patient, x_medicine, edge_index_pm, edge_index_mp,
           edge_label_index, W1l_pm, W1r_pm, b1_pm, W1l_mp, W1r_mp, b1_mp,
           W2l_pm, W2r_pm, b2_pm, W2l_mp, W2r_mp, b2_mp, Wd1, bd1, Wd2, bd2):
    i32 = jnp.int32
    src_pm = edge_index_pm[0].astype(i32)
    dst_pm = edge_index_pm[1].astype(i32)
    src_mp = edge_index_mp[0].astype(i32)
    dst_mp = edge_index_mp[1].astype(i32)
    row = edge_label_index[0].astype(i32)
    col = edge_label_index[1].astype(i32)

    zblk = jnp.zeros((_ZROWS, HALF), f32)
    oneblk = jnp.ones((_CH, HALF), f32)

    _segsum = _make_segsum()
    _counts = _make_counts()
    _edge_gather = _make_edge_gather()

    dst2 = jnp.concatenate([dst_pm, dst_mp])
    (Cboth,) = _counts(dst2, zblk, oneblk)
    cnt_pm = Cboth[0, :, 0:1]
    cnt_mp = Cboth[1, :, 0:1]

    (S1pm,) = _segsum(x_patient.reshape(2 * N_NODE, HALF), src_pm, dst_pm,
                      zblk)
    (S1mp,) = _segsum(x_medicine.reshape(2 * N_NODE, HALF), src_mp, dst_mp,
                      zblk)

    h_m = _fused_relu(S1pm, S1pm, cnt_pm, x_medicine, W1l_pm, W1r_pm,
                      b1_pm.reshape(1, D))
    h_p = _fused_relu(S1mp, S1mp, cnt_mp, x_patient, W1l_mp, W1r_mp,
                      b1_mp.reshape(1, D))

    (S2pm,) = _segsum(h_p.reshape(2 * N_NODE, HALF), src_pm, dst_pm, zblk)
    (S2mp,) = _segsum(h_m.reshape(2 * N_NODE, HALF), src_mp, dst_mp, zblk)

    M1 = _fused_post(S2pm, S2pm, cnt_pm, h_m, W2l_pm, W2r_pm,
                     b2_pm.reshape(1, D), Wd1[D:, :])
    P1 = _fused_post(S2mp, S2mp, cnt_mp, h_p, W2l_mp, W2r_mp,
                     b2_mp.reshape(1, D), Wd1[:D, :])

    Pg, Mg = _edge_gather(P1, M1, row, col)
    return _decoder(Pg, Mg, bd1.reshape(1, D), Wd2, bd2.reshape(1, 1))


# speedup vs baseline: 3.8323x; 1.4008x over previous
import functools

import jax
import jax.numpy as jnp
from jax import lax
from jax.experimental import pallas as pl
from jax.experimental.pallas import tpu as pltpu
from jax.experimental.pallas import tpu_sc as plsc

N_NODE = 10000
E_EDGE = 160000
L_LAB = 20000
D = 256
HALF = 128

_CH = 80
_NCHUNK = E_EDGE // _CH
_ZROWS = 80
_NROWCH = N_NODE // _ZROWS

f32 = jnp.float32


@functools.lru_cache(maxsize=None)
def _get_mesh():
    return plsc.VectorSubcoreMesh(core_axis_name="c", subcore_axis_name="s")


_NBUF = 4
_NCP = _NCHUNK // 16


@functools.lru_cache(maxsize=None)
def _make_segsum():
    @functools.partial(
        pl.kernel,
        out_type=[jax.ShapeDtypeStruct((2, N_NODE, HALF), f32)],
        mesh=_get_mesh(),
        scratch_types=(
            [pltpu.VMEM((_CH,), jnp.int32)] * _NBUF
            + [pltpu.VMEM((_CH,), jnp.int32)] * _NBUF
            + [pltpu.VMEM((_CH, HALF), f32)] * _NBUF
            + [
                pltpu.VMEM_SHARED((N_NODE, HALF), f32),
            ]
            + [pltpu.SemaphoreType.DMA] * _NBUF
        ),
    )
    def segsum(xs, gidx_hbm, dst_hbm, zblk, sums_hbm, *scr):
        gs = scr[:_NBUF]
        ds_ = scr[_NBUF:2 * _NBUF]
        rs = scr[2 * _NBUF:3 * _NBUF]
        acc_sh = scr[3 * _NBUF]
        sems = scr[3 * _NBUF + 1:]
        c = lax.axis_index("c")
        s = lax.axis_index("s")
        for j in range(8):
            ch = s + j * 16

            @pl.when(ch < _NROWCH)
            def _():
                pltpu.sync_copy(zblk, acc_sh.at[pl.ds(ch * _ZROWS, _ZROWS)])
        for b in range(_NBUF):
            base = (b * 16 + s) * _CH
            pltpu.sync_copy(gidx_hbm.at[pl.ds(c * E_EDGE + base, _CH)], gs[b])
            pltpu.sync_copy(dst_hbm.at[pl.ds(base, _CH)], ds_[b])
            pltpu.async_copy(xs.at[gs[b]], rs[b], sems[b])
        plsc.subcore_barrier()

        def grp(gi, carry):
            for b in range(_NBUF):
                j = gi * _NBUF + b

                @pl.when(j < _NCP)
                def _():
                    pltpu.make_async_copy(xs.at[gs[b]], rs[b],
                                          sems[b]).wait()
                    pltpu.sync_copy(rs[b], acc_sh.at[ds_[b]], add=True)
                    nj = j + _NBUF

                    @pl.when(nj < _NCP)
                    def _():
                        nbase = (nj * 16 + s) * _CH
                        pltpu.sync_copy(
                            gidx_hbm.at[pl.ds(c * E_EDGE + nbase, _CH)],
                            gs[b])
                        pltpu.sync_copy(dst_hbm.at[pl.ds(nbase, _CH)],
                                        ds_[b])
                        pltpu.async_copy(xs.at[gs[b]], rs[b], sems[b])
            return carry

        lax.fori_loop(0, (_NCP + _NBUF - 1) // _NBUF, grp, 0)
        plsc.subcore_barrier()
        for j in range(8):
            ch = s + j * 16

            @pl.when(ch < _NROWCH)
            def _():
                r0 = ch * _ZROWS
                pltpu.sync_copy(acc_sh.at[pl.ds(r0, _ZROWS)],
                                sums_hbm.at[c, pl.ds(r0, _ZROWS)])

    return segsum


@functools.lru_cache(maxsize=None)
def _make_counts():
    @functools.partial(
        pl.kernel,
        out_type=[jax.ShapeDtypeStruct((2, N_NODE, HALF), f32)],
        mesh=_get_mesh(),
        scratch_types=[
            pltpu.VMEM((_CH,), jnp.int32),
            pltpu.VMEM((_CH, HALF), f32),
            pltpu.VMEM((_ZROWS, HALF), f32),
            pltpu.VMEM_SHARED((N_NODE, HALF), f32),
        ],
    )
    def counts(dst2_hbm, zblk, oneblk, cnts_hbm, dst_v, ones_v, zrow_v,
               cnt_sh):
        c = lax.axis_index("c")
        s = lax.axis_index("s")
        pltpu.sync_copy(zblk, zrow_v)
        pltpu.sync_copy(oneblk, ones_v)
        for j in range(8):
            ch = s + j * 16

            @pl.when(ch < _NROWCH)
            def _():
                pltpu.sync_copy(zrow_v, cnt_sh.at[pl.ds(ch * _ZROWS, _ZROWS)])
        plsc.subcore_barrier()

        def step(i, carry):
            base = c * E_EDGE + (i * 16 + s) * _CH
            pltpu.sync_copy(dst2_hbm.at[pl.ds(base, _CH)], dst_v)
            pltpu.sync_copy(ones_v, cnt_sh.at[dst_v], add=True)
            return carry

        lax.fori_loop(0, _NCHUNK // 16, step, 0)
        plsc.subcore_barrier()
        for j in range(8):
            ch = s + j * 16

            @pl.when(ch < _NROWCH)
            def _():
                r0 = ch * _ZROWS
                pltpu.sync_copy(cnt_sh.at[pl.ds(r0, _ZROWS)],
                                cnts_hbm.at[c, pl.ds(r0, _ZROWS)])

    return counts


_GCH = 80
_NGCH = L_LAB // _GCH


@functools.lru_cache(maxsize=None)
def _make_edge_gather():
    @functools.partial(
        pl.kernel,
        out_type=[jax.ShapeDtypeStruct((L_LAB, D), f32),
                  jax.ShapeDtypeStruct((L_LAB, D), f32)],
        mesh=_get_mesh(),
        scratch_types=[
            pltpu.VMEM((_GCH,), jnp.int32),
            pltpu.VMEM((_GCH,), jnp.int32),
            pltpu.VMEM((_GCH, D), f32),
            pltpu.VMEM((_GCH, D), f32),
            pltpu.SemaphoreType.DMA,
        ],
    )
    def edge_gather(p1, m1, row_hbm, col_hbm, pg_hbm, mg_hbm,
                    idx_v, idx2_v, rows_v, rows2_v, sem):
        c = lax.axis_index("c")
        s = lax.axis_index("s")
        wid = s * 2 + c

        def step(i, carry):
            ch = wid + i * 32

            @pl.when(ch < _NGCH)
            def _():
                base = ch * _GCH
                pltpu.sync_copy(row_hbm.at[pl.ds(base, _GCH)], idx_v)
                pltpu.async_copy(p1.at[idx_v], rows_v, sem).wait()
                pltpu.sync_copy(rows_v, pg_hbm.at[pl.ds(base, _GCH)])
                pltpu.sync_copy(col_hbm.at[pl.ds(base, _GCH)], idx2_v)
                pltpu.async_copy(m1.at[idx2_v], rows2_v, sem).wait()
                pltpu.sync_copy(rows2_v, mg_hbm.at[pl.ds(base, _GCH)])
            return carry

        lax.fori_loop(0, (_NGCH + 31) // 32, step, 0)

    return edge_gather


_BM = 1000


def _make_fused(relu: bool, with_post: bool):
    def body(*refs):
        if with_post:
            (sa_ref, sb_ref, cnt_ref, x_ref, wl_ref, wr_ref, b_ref, wp_ref,
             out_ref) = refs
        else:
            (sa_ref, sb_ref, cnt_ref, x_ref, wl_ref, wr_ref, b_ref,
             out_ref) = refs
        inv = 1.0 / jnp.maximum(cnt_ref[...], 1.0)
        wl = wl_ref[...]
        wr = wr_ref[...]
        b = b_ref[...]
        if with_post:
            wp = wp_ref[...]
            wl = jnp.dot(wl, wp, preferred_element_type=f32)
            wr = jnp.dot(wr, wp, preferred_element_type=f32)
            b = jnp.dot(b, wp, preferred_element_type=f32)
        acc = (jnp.dot(sa_ref[0] * inv, wl[:HALF, :],
                       preferred_element_type=f32)
               + jnp.dot(sb_ref[0] * inv, wl[HALF:, :],
                         preferred_element_type=f32)
               + jnp.dot(x_ref[...], wr, preferred_element_type=f32) + b)
        if relu:
            acc = jnp.maximum(acc, 0.0)
        out_ref[...] = acc

    n_rows = N_NODE // _BM
    in_specs = [
        pl.BlockSpec((1, _BM, HALF), lambda i: (0, i, 0)),
        pl.BlockSpec((1, _BM, HALF), lambda i: (1, i, 0)),
        pl.BlockSpec((_BM, 1), lambda i: (i, 0)),
        pl.BlockSpec((_BM, D), lambda i: (i, 0)),
        pl.BlockSpec((D, D), lambda i: (0, 0)),
        pl.BlockSpec((D, D), lambda i: (0, 0)),
        pl.BlockSpec((1, D), lambda i: (0, 0)),
    ]
    if with_post:
        in_specs.append(pl.BlockSpec((D, D), lambda i: (0, 0)))
    return pl.pallas_call(
        body,
        grid=(n_rows,),
        in_specs=in_specs,
        out_specs=pl.BlockSpec((_BM, D), lambda i: (i, 0)),
        out_shape=jax.ShapeDtypeStruct((N_NODE, D), f32),
    )


_fused_relu = _make_fused(True, False)
_fused_post = _make_fused(False, True)

_BL = 2000


def _dec_body(pg_ref, mg_ref, bd1_ref, wd2_ref, bd2_ref, out_ref):
    z = jnp.maximum(pg_ref[...] + mg_ref[...] + bd1_ref[...], 0.0)
    out_ref[...] = jnp.dot(z, wd2_ref[...], preferred_element_type=f32) + bd2_ref[...]


_decoder = pl.pallas_call(
    _dec_body,
    grid=(L_LAB // _BL,),
    in_specs=[
        pl.BlockSpec((_BL, D), lambda i: (i, 0)),
        pl.BlockSpec((_BL, D), lambda i: (i, 0)),
        pl.BlockSpec((1, D), lambda i: (0, 0)),
        pl.BlockSpec((D, 1), lambda i: (0, 0)),
        pl.BlockSpec((1, 1), lambda i: (0, 0)),
    ],
    out_specs=pl.BlockSpec((_BL, 1), lambda i: (i, 0)),
    out_shape=jax.ShapeDtypeStruct((L_LAB, 1), f32),
)


def kernel(x_patient, x_medicine, edge_index_pm, edge_index_mp,
           edge_label_index, W1l_pm, W1r_pm, b1_pm, W1l_mp, W1r_mp, b1_mp,
           W2l_pm, W2r_pm, b2_pm, W2l_mp, W2r_mp, b2_mp, Wd1, bd1, Wd2, bd2):
    i32 = jnp.int32
    src_pm = edge_index_pm[0].astype(i32)
    dst_pm = edge_index_pm[1].astype(i32)
    src_mp = edge_index_mp[0].astype(i32)
    dst_mp = edge_index_mp[1].astype(i32)
    row = edge_label_index[0].astype(i32)
    col = edge_label_index[1].astype(i32)

    zblk = jnp.zeros((_ZROWS, HALF), f32)
    oneblk = jnp.ones((_CH, HALF), f32)

    _segsum = _make_segsum()
    _counts = _make_counts()
    _edge_gather = _make_edge_gather()

    dst2 = jnp.concatenate([dst_pm, dst_mp])
    (Cboth,) = _counts(dst2, zblk, oneblk)
    cnt_pm = Cboth[0, :, 0:1]
    cnt_mp = Cboth[1, :, 0:1]

    gidx_pm = jnp.concatenate([src_pm * 2, src_pm * 2 + 1])
    gidx_mp = jnp.concatenate([src_mp * 2, src_mp * 2 + 1])

    (S1pm,) = _segsum(x_patient.reshape(2 * N_NODE, HALF), gidx_pm, dst_pm,
                      zblk)
    (S1mp,) = _segsum(x_medicine.reshape(2 * N_NODE, HALF), gidx_mp, dst_mp,
                      zblk)

    h_m = _fused_relu(S1pm, S1pm, cnt_pm, x_medicine, W1l_pm, W1r_pm,
                      b1_pm.reshape(1, D))
    h_p = _fused_relu(S1mp, S1mp, cnt_mp, x_patient, W1l_mp, W1r_mp,
                      b1_mp.reshape(1, D))

    (S2pm,) = _segsum(h_p.reshape(2 * N_NODE, HALF), gidx_pm, dst_pm, zblk)
    (S2mp,) = _segsum(h_m.reshape(2 * N_NODE, HALF), gidx_mp, dst_mp, zblk)

    M1 = _fused_post(S2pm, S2pm, cnt_pm, h_m, W2l_pm, W2r_pm,
                     b2_pm.reshape(1, D), Wd1[D:, :])
    P1 = _fused_post(S2mp, S2mp, cnt_mp, h_p, W2l_mp, W2r_mp,
                     b2_mp.reshape(1, D), Wd1[:D, :])

    Pg, Mg = _edge_gather(P1, M1, row, col)
    return _decoder(Pg, Mg, bd1.reshape(1, D), Wd2, bd2.reshape(1, 1))


# speedup vs baseline: 5.6306x; 1.4692x over previous
import functools

import jax
import jax.numpy as jnp
from jax import lax
from jax.experimental import pallas as pl
from jax.experimental.pallas import tpu as pltpu
from jax.experimental.pallas import tpu_sc as plsc

N_NODE = 10000
E_EDGE = 160000
L_LAB = 20000
D = 256
HALF = 128

_CH = 80
_NCHUNK = E_EDGE // _CH
_ZROWS = 80
_NROWCH = N_NODE // _ZROWS

f32 = jnp.float32


@functools.lru_cache(maxsize=None)
def _get_mesh():
    return plsc.VectorSubcoreMesh(core_axis_name="c", subcore_axis_name="s")


_NBUF = 4
_IBUF = 2 * _NBUF
_NCP = _NCHUNK // 16


@functools.lru_cache(maxsize=None)
def _make_segsum():
    @functools.partial(
        pl.kernel,
        out_type=[jax.ShapeDtypeStruct((2, N_NODE, HALF), f32)],
        mesh=_get_mesh(),
        scratch_types=(
            [pltpu.VMEM((_CH,), jnp.int32)] * _IBUF
            + [pltpu.VMEM((_CH,), jnp.int32)] * _IBUF
            + [pltpu.VMEM((_CH, HALF), f32)] * _NBUF
            + [
                pltpu.VMEM_SHARED((N_NODE, HALF), f32),
            ]
            + [pltpu.SemaphoreType.DMA] * _NBUF
            + [pltpu.SemaphoreType.DMA] * _IBUF
        ),
    )
    def segsum(xs, gidx_hbm, dst_hbm, zblk, sums_hbm, *scr):
        gs = scr[:_IBUF]
        ds_ = scr[_IBUF:2 * _IBUF]
        rs = scr[2 * _IBUF:2 * _IBUF + _NBUF]
        acc_sh = scr[2 * _IBUF + _NBUF]
        gsem = scr[2 * _IBUF + _NBUF + 1:2 * _IBUF + 2 * _NBUF + 1]
        isem = scr[2 * _IBUF + 2 * _NBUF + 1:]
        c = lax.axis_index("c")
        s = lax.axis_index("s")

        def fire_idx(j, b):
            base = (j * 16 + s) * _CH
            pltpu.async_copy(gidx_hbm.at[pl.ds(c * E_EDGE + base, _CH)],
                             gs[b], isem[b])
            pltpu.async_copy(dst_hbm.at[pl.ds(base, _CH)], ds_[b], isem[b])

        def drain_idx(b):
            pltpu.make_async_copy(gidx_hbm.at[pl.ds(0, _CH)], gs[b],
                                  isem[b]).wait()
            pltpu.make_async_copy(dst_hbm.at[pl.ds(0, _CH)], ds_[b],
                                  isem[b]).wait()

        GLEAD = _NBUF - 1
        ILEAD = _IBUF - 1

        for b in range(ILEAD):
            fire_idx(b, b)
        for j in range(8):
            ch = s + j * 16

            @pl.when(ch < _NROWCH)
            def _():
                pltpu.sync_copy(zblk, acc_sh.at[pl.ds(ch * _ZROWS, _ZROWS)])
        for b in range(GLEAD):
            drain_idx(b)
            pltpu.async_copy(xs.at[gs[b]], rs[b], gsem[b])
        plsc.subcore_barrier()

        def grp(gi, carry):
            for b8 in range(_IBUF):
                j = gi * _IBUF + b8
                rb = b8 % _NBUF

                @pl.when(j < _NCP)
                def _():
                    pltpu.make_async_copy(xs.at[gs[b8]], rs[rb],
                                          gsem[rb]).wait()
                    pltpu.sync_copy(rs[rb], acc_sh.at[ds_[b8]], add=True)
                    nj8 = j + ILEAD

                    @pl.when(nj8 < _NCP)
                    def _():
                        fire_idx(nj8, (b8 + ILEAD) % _IBUF)
                    nj = j + GLEAD

                    @pl.when(nj < _NCP)
                    def _():
                        ib = (b8 + GLEAD) % _IBUF
                        drain_idx(ib)
                        pltpu.async_copy(xs.at[gs[ib]], rs[(rb + GLEAD) % _NBUF],
                                        gsem[(rb + GLEAD) % _NBUF])
            return carry

        lax.fori_loop(0, (_NCP + _IBUF - 1) // _IBUF, grp, 0)
        plsc.subcore_barrier()
        for j in range(8):
            ch = s + j * 16

            @pl.when(ch < _NROWCH)
            def _():
                r0 = ch * _ZROWS
                pltpu.sync_copy(acc_sh.at[pl.ds(r0, _ZROWS)],
                                sums_hbm.at[c, pl.ds(r0, _ZROWS)])

    return segsum


@functools.lru_cache(maxsize=None)
def _make_counts():
    @functools.partial(
        pl.kernel,
        out_type=[jax.ShapeDtypeStruct((2, N_NODE, HALF), f32)],
        mesh=_get_mesh(),
        scratch_types=[
            pltpu.VMEM((_CH,), jnp.int32),
            pltpu.VMEM((_CH, HALF), f32),
            pltpu.VMEM((_ZROWS, HALF), f32),
            pltpu.VMEM_SHARED((N_NODE, HALF), f32),
        ],
    )
    def counts(dst2_hbm, zblk, oneblk, cnts_hbm, dst_v, ones_v, zrow_v,
               cnt_sh):
        c = lax.axis_index("c")
        s = lax.axis_index("s")
        pltpu.sync_copy(zblk, zrow_v)
        pltpu.sync_copy(oneblk, ones_v)
        for j in range(8):
            ch = s + j * 16

            @pl.when(ch < _NROWCH)
            def _():
                pltpu.sync_copy(zrow_v, cnt_sh.at[pl.ds(ch * _ZROWS, _ZROWS)])
        plsc.subcore_barrier()

        def step(i, carry):
            base = c * E_EDGE + (i * 16 + s) * _CH
            pltpu.sync_copy(dst2_hbm.at[pl.ds(base, _CH)], dst_v)
            pltpu.sync_copy(ones_v, cnt_sh.at[dst_v], add=True)
            return carry

        lax.fori_loop(0, _NCHUNK // 16, step, 0)
        plsc.subcore_barrier()
        for j in range(8):
            ch = s + j * 16

            @pl.when(ch < _NROWCH)
            def _():
                r0 = ch * _ZROWS
                pltpu.sync_copy(cnt_sh.at[pl.ds(r0, _ZROWS)],
                                cnts_hbm.at[c, pl.ds(r0, _ZROWS)])

    return counts


_GCH = 80
_NGCH = L_LAB // _GCH


@functools.lru_cache(maxsize=None)
def _make_edge_gather():
    @functools.partial(
        pl.kernel,
        out_type=[jax.ShapeDtypeStruct((L_LAB, D), f32),
                  jax.ShapeDtypeStruct((L_LAB, D), f32)],
        mesh=_get_mesh(),
        scratch_types=[
            pltpu.VMEM((_GCH,), jnp.int32),
            pltpu.VMEM((_GCH,), jnp.int32),
            pltpu.VMEM((_GCH, D), f32),
            pltpu.VMEM((_GCH, D), f32),
            pltpu.SemaphoreType.DMA,
        ],
    )
    def edge_gather(p1, m1, row_hbm, col_hbm, pg_hbm, mg_hbm,
                    idx_v, idx2_v, rows_v, rows2_v, sem):
        c = lax.axis_index("c")
        s = lax.axis_index("s")
        wid = s * 2 + c

        def step(i, carry):
            ch = wid + i * 32

            @pl.when(ch < _NGCH)
            def _():
                base = ch * _GCH
                pltpu.sync_copy(row_hbm.at[pl.ds(base, _GCH)], idx_v)
                pltpu.async_copy(p1.at[idx_v], rows_v, sem).wait()
                pltpu.sync_copy(rows_v, pg_hbm.at[pl.ds(base, _GCH)])
                pltpu.sync_copy(col_hbm.at[pl.ds(base, _GCH)], idx2_v)
                pltpu.async_copy(m1.at[idx2_v], rows2_v, sem).wait()
                pltpu.sync_copy(rows2_v, mg_hbm.at[pl.ds(base, _GCH)])
            return carry

        lax.fori_loop(0, (_NGCH + 31) // 32, step, 0)

    return edge_gather


_BM = 1000


def _make_fused(relu: bool, with_post: bool):
    def body(*refs):
        if with_post:
            (sa_ref, sb_ref, cnt_ref, x_ref, wl_ref, wr_ref, b_ref, wp_ref,
             out_ref) = refs
        else:
            (sa_ref, sb_ref, cnt_ref, x_ref, wl_ref, wr_ref, b_ref,
             out_ref) = refs
        inv = 1.0 / jnp.maximum(cnt_ref[...], 1.0)
        wl = wl_ref[...]
        wr = wr_ref[...]
        b = b_ref[...]
        if with_post:
            wp = wp_ref[...]
            wl = jnp.dot(wl, wp, preferred_element_type=f32)
            wr = jnp.dot(wr, wp, preferred_element_type=f32)
            b = jnp.dot(b, wp, preferred_element_type=f32)
        acc = (jnp.dot(sa_ref[0] * inv, wl[:HALF, :],
                       preferred_element_type=f32)
               + jnp.dot(sb_ref[0] * inv, wl[HALF:, :],
                         preferred_element_type=f32)
               + jnp.dot(x_ref[...], wr, preferred_element_type=f32) + b)
        if relu:
            acc = jnp.maximum(acc, 0.0)
        out_ref[...] = acc

    n_rows = N_NODE // _BM
    in_specs = [
        pl.BlockSpec((1, _BM, HALF), lambda i: (0, i, 0)),
        pl.BlockSpec((1, _BM, HALF), lambda i: (1, i, 0)),
        pl.BlockSpec((_BM, 1), lambda i: (i, 0)),
        pl.BlockSpec((_BM, D), lambda i: (i, 0)),
        pl.BlockSpec((D, D), lambda i: (0, 0)),
        pl.BlockSpec((D, D), lambda i: (0, 0)),
        pl.BlockSpec((1, D), lambda i: (0, 0)),
    ]
    if with_post:
        in_specs.append(pl.BlockSpec((D, D), lambda i: (0, 0)))
    return pl.pallas_call(
        body,
        grid=(n_rows,),
        in_specs=in_specs,
        out_specs=pl.BlockSpec((_BM, D), lambda i: (i, 0)),
        out_shape=jax.ShapeDtypeStruct((N_NODE, D), f32),
    )


_fused_relu = _make_fused(True, False)
_fused_post = _make_fused(False, True)

_BL = 2000


def _dec_body(pg_ref, mg_ref, bd1_ref, wd2_ref, bd2_ref, out_ref):
    z = jnp.maximum(pg_ref[...] + mg_ref[...] + bd1_ref[...], 0.0)
    out_ref[...] = jnp.dot(z, wd2_ref[...], preferred_element_type=f32) + bd2_ref[...]


_decoder = pl.pallas_call(
    _dec_body,
    grid=(L_LAB // _BL,),
    in_specs=[
        pl.BlockSpec((_BL, D), lambda i: (i, 0)),
        pl.BlockSpec((_BL, D), lambda i: (i, 0)),
        pl.BlockSpec((1, D), lambda i: (0, 0)),
        pl.BlockSpec((D, 1), lambda i: (0, 0)),
        pl.BlockSpec((1, 1), lambda i: (0, 0)),
    ],
    out_specs=pl.BlockSpec((_BL, 1), lambda i: (i, 0)),
    out_shape=jax.ShapeDtypeStruct((L_LAB, 1), f32),
)


def kernel(x_patient, x_medicine, edge_index_pm, edge_index_mp,
           edge_label_index, W1l_pm, W1r_pm, b1_pm, W1l_mp, W1r_mp, b1_mp,
           W2l_pm, W2r_pm, b2_pm, W2l_mp, W2r_mp, b2_mp, Wd1, bd1, Wd2, bd2):
    i32 = jnp.int32
    src_pm = edge_index_pm[0].astype(i32)
    dst_pm = edge_index_pm[1].astype(i32)
    src_mp = edge_index_mp[0].astype(i32)
    dst_mp = edge_index_mp[1].astype(i32)
    row = edge_label_index[0].astype(i32)
    col = edge_label_index[1].astype(i32)

    zblk = jnp.zeros((_ZROWS, HALF), f32)
    oneblk = jnp.ones((_CH, HALF), f32)

    _segsum = _make_segsum()
    _counts = _make_counts()
    _edge_gather = _make_edge_gather()

    dst2 = jnp.concatenate([dst_pm, dst_mp])
    (Cboth,) = _counts(dst2, zblk, oneblk)
    cnt_pm = Cboth[0, :, 0:1]
    cnt_mp = Cboth[1, :, 0:1]

    gidx_pm = jnp.concatenate([src_pm * 2, src_pm * 2 + 1])
    gidx_mp = jnp.concatenate([src_mp * 2, src_mp * 2 + 1])

    (S1pm,) = _segsum(x_patient.reshape(2 * N_NODE, HALF), gidx_pm, dst_pm,
                      zblk)
    (S1mp,) = _segsum(x_medicine.reshape(2 * N_NODE, HALF), gidx_mp, dst_mp,
                      zblk)

    h_m = _fused_relu(S1pm, S1pm, cnt_pm, x_medicine, W1l_pm, W1r_pm,
                      b1_pm.reshape(1, D))
    h_p = _fused_relu(S1mp, S1mp, cnt_mp, x_patient, W1l_mp, W1r_mp,
                      b1_mp.reshape(1, D))

    (S2pm,) = _segsum(h_p.reshape(2 * N_NODE, HALF), gidx_pm, dst_pm, zblk)
    (S2mp,) = _segsum(h_m.reshape(2 * N_NODE, HALF), gidx_mp, dst_mp, zblk)

    M1 = _fused_post(S2pm, S2pm, cnt_pm, h_m, W2l_pm, W2r_pm,
                     b2_pm.reshape(1, D), Wd1[D:, :])
    P1 = _fused_post(S2mp, S2mp, cnt_mp, h_p, W2l_mp, W2r_mp,
                     b2_mp.reshape(1, D), Wd1[:D, :])

    Pg, Mg = _edge_gather(P1, M1, row, col)
    return _decoder(Pg, Mg, bd1.reshape(1, D), Wd2, bd2.reshape(1, 1))


# speedup vs baseline: 5.9306x; 1.0533x over previous
import functools

import jax
import jax.numpy as jnp
from jax import lax
from jax.experimental import pallas as pl
from jax.experimental.pallas import tpu as pltpu
from jax.experimental.pallas import tpu_sc as plsc

N_NODE = 10000
E_EDGE = 160000
L_LAB = 20000
D = 256
HALF = 128

_CH = 80
_NCHUNK = E_EDGE // _CH
_ZROWS = 80
_NROWCH = N_NODE // _ZROWS

f32 = jnp.float32


@functools.lru_cache(maxsize=None)
def _get_mesh():
    return plsc.VectorSubcoreMesh(core_axis_name="c", subcore_axis_name="s")


_NBUF = 4
_IBUF = 2 * _NBUF
_NCP = _NCHUNK // 16


@functools.lru_cache(maxsize=None)
def _make_segsum():
    @functools.partial(
        pl.kernel,
        out_type=[jax.ShapeDtypeStruct((2, N_NODE, HALF), f32)],
        mesh=_get_mesh(),
        scratch_types=(
            [pltpu.VMEM((_CH,), jnp.int32)] * _IBUF
            + [pltpu.VMEM((_CH,), jnp.int32)] * _IBUF
            + [pltpu.VMEM((_CH, HALF), f32)] * _NBUF
            + [
                pltpu.VMEM_SHARED((N_NODE, HALF), f32),
            ]
            + [pltpu.SemaphoreType.DMA] * _NBUF
            + [pltpu.SemaphoreType.DMA] * _IBUF
        ),
    )
    def segsum(xs, gidx_hbm, dst_hbm, zblk, sums_hbm, *scr):
        gs = scr[:_IBUF]
        ds_ = scr[_IBUF:2 * _IBUF]
        rs = scr[2 * _IBUF:2 * _IBUF + _NBUF]
        acc_sh = scr[2 * _IBUF + _NBUF]
        gsem = scr[2 * _IBUF + _NBUF + 1:2 * _IBUF + 2 * _NBUF + 1]
        isem = scr[2 * _IBUF + 2 * _NBUF + 1:]
        c = lax.axis_index("c")
        s = lax.axis_index("s")

        def fire_idx(j, b):
            base = (j * 16 + s) * _CH
            pltpu.async_copy(gidx_hbm.at[pl.ds(c * E_EDGE + base, _CH)],
                             gs[b], isem[b])
            pltpu.async_copy(dst_hbm.at[pl.ds(base, _CH)], ds_[b], isem[b])

        def drain_idx(b):
            pltpu.make_async_copy(gidx_hbm.at[pl.ds(0, _CH)], gs[b],
                                  isem[b]).wait()
            pltpu.make_async_copy(dst_hbm.at[pl.ds(0, _CH)], ds_[b],
                                  isem[b]).wait()

        GLEAD = _NBUF - 1
        ILEAD = _IBUF - 1

        for b in range(ILEAD):
            fire_idx(b, b)
        for j in range(8):
            ch = s + j * 16

            @pl.when(ch < _NROWCH)
            def _():
                pltpu.sync_copy(zblk, acc_sh.at[pl.ds(ch * _ZROWS, _ZROWS)])
        for b in range(GLEAD):
            drain_idx(b)
            pltpu.async_copy(xs.at[gs[b]], rs[b], gsem[b])
        plsc.subcore_barrier()

        def grp(gi, carry):
            for b8 in range(_IBUF):
                j = gi * _IBUF + b8
                rb = b8 % _NBUF

                @pl.when(j < _NCP)
                def _():
                    pltpu.make_async_copy(xs.at[gs[b8]], rs[rb],
                                          gsem[rb]).wait()
                    pltpu.sync_copy(rs[rb], acc_sh.at[ds_[b8]], add=True)
                    nj8 = j + ILEAD

                    @pl.when(nj8 < _NCP)
                    def _():
                        fire_idx(nj8, (b8 + ILEAD) % _IBUF)
                    nj = j + GLEAD

                    @pl.when(nj < _NCP)
                    def _():
                        ib = (b8 + GLEAD) % _IBUF
                        drain_idx(ib)
                        pltpu.async_copy(xs.at[gs[ib]], rs[(rb + GLEAD) % _NBUF],
                                        gsem[(rb + GLEAD) % _NBUF])
            return carry

        lax.fori_loop(0, (_NCP + _IBUF - 1) // _IBUF, grp, 0)
        plsc.subcore_barrier()
        for j in range(8):
            ch = s + j * 16

            @pl.when(ch < _NROWCH)
            def _():
                r0 = ch * _ZROWS
                pltpu.sync_copy(acc_sh.at[pl.ds(r0, _ZROWS)],
                                sums_hbm.at[c, pl.ds(r0, _ZROWS)])

    return segsum


_CW = HALF


@functools.lru_cache(maxsize=None)
def _make_counts():
    @functools.partial(
        pl.kernel,
        out_type=[jax.ShapeDtypeStruct((2, N_NODE, _CW), f32)],
        mesh=_get_mesh(),
        scratch_types=(
            [pltpu.VMEM((_CH,), jnp.int32)] * _IBUF
            + [
                pltpu.VMEM((_CH, _CW), f32),
                pltpu.VMEM_SHARED((N_NODE, _CW), f32),
            ]
            + [pltpu.SemaphoreType.DMA] * _IBUF
        ),
    )
    def counts(dst2_hbm, zblk, oneblk, cnts_hbm, *scr):
        ds_ = scr[:_IBUF]
        ones_v = scr[_IBUF]
        cnt_sh = scr[_IBUF + 1]
        isem = scr[_IBUF + 2:]
        c = lax.axis_index("c")
        s = lax.axis_index("s")

        def fire_idx(j, b):
            base = c * E_EDGE + (j * 16 + s) * _CH
            pltpu.async_copy(dst2_hbm.at[pl.ds(base, _CH)], ds_[b], isem[b])

        def drain_idx(b):
            pltpu.make_async_copy(dst2_hbm.at[pl.ds(0, _CH)], ds_[b],
                                  isem[b]).wait()

        ILEAD = _IBUF - 1
        for b in range(ILEAD):
            fire_idx(b, b)
        pltpu.sync_copy(oneblk, ones_v)
        for j in range(8):
            ch = s + j * 16

            @pl.when(ch < _NROWCH)
            def _():
                pltpu.sync_copy(zblk, cnt_sh.at[pl.ds(ch * _ZROWS, _ZROWS)])
        plsc.subcore_barrier()

        def grp(gi, carry):
            for b8 in range(_IBUF):
                j = gi * _IBUF + b8

                @pl.when(j < _NCP)
                def _():
                    drain_idx(b8)
                    pltpu.sync_copy(ones_v, cnt_sh.at[ds_[b8]], add=True)
                    nj = j + ILEAD

                    @pl.when(nj < _NCP)
                    def _():
                        fire_idx(nj, (b8 + ILEAD) % _IBUF)
            return carry

        lax.fori_loop(0, (_NCP + _IBUF - 1) // _IBUF, grp, 0)
        plsc.subcore_barrier()
        for j in range(8):
            ch = s + j * 16

            @pl.when(ch < _NROWCH)
            def _():
                r0 = ch * _ZROWS
                pltpu.sync_copy(cnt_sh.at[pl.ds(r0, _ZROWS)],
                                cnts_hbm.at[c, pl.ds(r0, _ZROWS)])

    return counts


_GCH = 80
_NGCH = L_LAB // _GCH


@functools.lru_cache(maxsize=None)
def _make_edge_gather():
    @functools.partial(
        pl.kernel,
        out_type=[jax.ShapeDtypeStruct((L_LAB, D), f32),
                  jax.ShapeDtypeStruct((L_LAB, D), f32)],
        mesh=_get_mesh(),
        scratch_types=[
            pltpu.VMEM((_GCH,), jnp.int32),
            pltpu.VMEM((_GCH,), jnp.int32),
            pltpu.VMEM((_GCH, D), f32),
            pltpu.VMEM((_GCH, D), f32),
            pltpu.SemaphoreType.DMA,
        ],
    )
    def edge_gather(p1, m1, row_hbm, col_hbm, pg_hbm, mg_hbm,
                    idx_v, idx2_v, rows_v, rows2_v, sem):
        c = lax.axis_index("c")
        s = lax.axis_index("s")
        wid = s * 2 + c

        def step(i, carry):
            ch = wid + i * 32

            @pl.when(ch < _NGCH)
            def _():
                base = ch * _GCH
                pltpu.sync_copy(row_hbm.at[pl.ds(base, _GCH)], idx_v)
                pltpu.async_copy(p1.at[idx_v], rows_v, sem).wait()
                pltpu.sync_copy(rows_v, pg_hbm.at[pl.ds(base, _GCH)])
                pltpu.sync_copy(col_hbm.at[pl.ds(base, _GCH)], idx2_v)
                pltpu.async_copy(m1.at[idx2_v], rows2_v, sem).wait()
                pltpu.sync_copy(rows2_v, mg_hbm.at[pl.ds(base, _GCH)])
            return carry

        lax.fori_loop(0, (_NGCH + 31) // 32, step, 0)

    return edge_gather


_BM = 1000


def _make_fused(relu: bool, with_post: bool):
    def body(*refs):
        if with_post:
            (sa_ref, sb_ref, cnt_ref, x_ref, wl_ref, wr_ref, b_ref, wp_ref,
             out_ref) = refs
        else:
            (sa_ref, sb_ref, cnt_ref, x_ref, wl_ref, wr_ref, b_ref,
             out_ref) = refs
        inv = 1.0 / jnp.maximum(cnt_ref[...], 1.0)
        wl = wl_ref[...]
        wr = wr_ref[...]
        b = b_ref[...]
        if with_post:
            wp = wp_ref[...]
            wl = jnp.dot(wl, wp, preferred_element_type=f32)
            wr = jnp.dot(wr, wp, preferred_element_type=f32)
            b = jnp.dot(b, wp, preferred_element_type=f32)
        acc = (jnp.dot(sa_ref[0] * inv, wl[:HALF, :],
                       preferred_element_type=f32)
               + jnp.dot(sb_ref[0] * inv, wl[HALF:, :],
                         preferred_element_type=f32)
               + jnp.dot(x_ref[...], wr, preferred_element_type=f32) + b)
        if relu:
            acc = jnp.maximum(acc, 0.0)
        out_ref[...] = acc

    n_rows = N_NODE // _BM
    in_specs = [
        pl.BlockSpec((1, _BM, HALF), lambda i: (0, i, 0)),
        pl.BlockSpec((1, _BM, HALF), lambda i: (1, i, 0)),
        pl.BlockSpec((_BM, 1), lambda i: (i, 0)),
        pl.BlockSpec((_BM, D), lambda i: (i, 0)),
        pl.BlockSpec((D, D), lambda i: (0, 0)),
        pl.BlockSpec((D, D), lambda i: (0, 0)),
        pl.BlockSpec((1, D), lambda i: (0, 0)),
    ]
    if with_post:
        in_specs.append(pl.BlockSpec((D, D), lambda i: (0, 0)))
    return pl.pallas_call(
        body,
        grid=(n_rows,),
        in_specs=in_specs,
        out_specs=pl.BlockSpec((_BM, D), lambda i: (i, 0)),
        out_shape=jax.ShapeDtypeStruct((N_NODE, D), f32),
    )


_fused_relu = _make_fused(True, False)
_fused_post = _make_fused(False, True)

_BL = 2000


def _dec_body(pg_ref, mg_ref, bd1_ref, wd2_ref, bd2_ref, out_ref):
    z = jnp.maximum(pg_ref[...] + mg_ref[...] + bd1_ref[...], 0.0)
    out_ref[...] = jnp.dot(z, wd2_ref[...], preferred_element_type=f32) + bd2_ref[...]


_decoder = pl.pallas_call(
    _dec_body,
    grid=(L_LAB // _BL,),
    in_specs=[
        pl.BlockSpec((_BL, D), lambda i: (i, 0)),
        pl.BlockSpec((_BL, D), lambda i: (i, 0)),
        pl.BlockSpec((1, D), lambda i: (0, 0)),
        pl.BlockSpec((D, 1), lambda i: (0, 0)),
        pl.BlockSpec((1, 1), lambda i: (0, 0)),
    ],
    out_specs=pl.BlockSpec((_BL, 1), lambda i: (i, 0)),
    out_shape=jax.ShapeDtypeStruct((L_LAB, 1), f32),
)


def kernel(x_patient, x_medicine, edge_index_pm, edge_index_mp,
           edge_label_index, W1l_pm, W1r_pm, b1_pm, W1l_mp, W1r_mp, b1_mp,
           W2l_pm, W2r_pm, b2_pm, W2l_mp, W2r_mp, b2_mp, Wd1, bd1, Wd2, bd2):
    i32 = jnp.int32
    src_pm = edge_index_pm[0].astype(i32)
    dst_pm = edge_index_pm[1].astype(i32)
    src_mp = edge_index_mp[0].astype(i32)
    dst_mp = edge_index_mp[1].astype(i32)
    row = edge_label_index[0].astype(i32)
    col = edge_label_index[1].astype(i32)

    zblk = jnp.zeros((_ZROWS, HALF), f32)
    zblk_c = jnp.zeros((_ZROWS, _CW), f32)
    oneblk = jnp.ones((_CH, _CW), f32)

    _segsum = _make_segsum()
    _counts = _make_counts()
    _edge_gather = _make_edge_gather()

    dst2 = jnp.concatenate([dst_pm, dst_mp])
    (Cboth,) = _counts(dst2, zblk_c, oneblk)
    cnt_pm = Cboth[0, :, 0:1]
    cnt_mp = Cboth[1, :, 0:1]

    gidx_pm = jnp.concatenate([src_pm * 2, src_pm * 2 + 1])
    gidx_mp = jnp.concatenate([src_mp * 2, src_mp * 2 + 1])

    (S1pm,) = _segsum(x_patient.reshape(2 * N_NODE, HALF), gidx_pm, dst_pm,
                      zblk)
    (S1mp,) = _segsum(x_medicine.reshape(2 * N_NODE, HALF), gidx_mp, dst_mp,
                      zblk)

    h_m = _fused_relu(S1pm, S1pm, cnt_pm, x_medicine, W1l_pm, W1r_pm,
                      b1_pm.reshape(1, D))
    h_p = _fused_relu(S1mp, S1mp, cnt_mp, x_patient, W1l_mp, W1r_mp,
                      b1_mp.reshape(1, D))

    (S2pm,) = _segsum(h_p.reshape(2 * N_NODE, HALF), gidx_pm, dst_pm, zblk)
    (S2mp,) = _segsum(h_m.reshape(2 * N_NODE, HALF), gidx_mp, dst_mp, zblk)

    M1 = _fused_post(S2pm, S2pm, cnt_pm, h_m, W2l_pm, W2r_pm,
                     b2_pm.reshape(1, D), Wd1[D:, :])
    P1 = _fused_post(S2mp, S2mp, cnt_mp, h_p, W2l_mp, W2r_mp,
                     b2_mp.reshape(1, D), Wd1[:D, :])

    Pg, Mg = _edge_gather(P1, M1, row, col)
    return _decoder(Pg, Mg, bd1.reshape(1, D), Wd2, bd2.reshape(1, 1))


# speedup vs baseline: 5.9658x; 1.0059x over previous
import functools

import jax
import jax.numpy as jnp
from jax import lax
from jax.experimental import pallas as pl
from jax.experimental.pallas import tpu as pltpu
from jax.experimental.pallas import tpu_sc as plsc

N_NODE = 10000
E_EDGE = 160000
L_LAB = 20000
D = 256
HALF = 128

_CH = 80
_NCHUNK = E_EDGE // _CH
_ZROWS = 80
_NROWCH = N_NODE // _ZROWS

f32 = jnp.float32


@functools.lru_cache(maxsize=None)
def _get_mesh():
    return plsc.VectorSubcoreMesh(core_axis_name="c", subcore_axis_name="s")


_NBUF = 4
_IBUF = 2 * _NBUF
_NCP = _NCHUNK // 16


@functools.lru_cache(maxsize=None)
def _make_segsum():
    @functools.partial(
        pl.kernel,
        out_type=[jax.ShapeDtypeStruct((2, N_NODE, HALF), f32)],
        mesh=_get_mesh(),
        scratch_types=(
            [pltpu.VMEM((_CH,), jnp.int32)] * _IBUF
            + [pltpu.VMEM((_CH,), jnp.int32)] * _IBUF
            + [pltpu.VMEM((_CH, HALF), f32)] * _NBUF
            + [
                pltpu.VMEM_SHARED((N_NODE, HALF), f32),
            ]
            + [pltpu.SemaphoreType.DMA] * _NBUF
            + [pltpu.SemaphoreType.DMA] * _IBUF
        ),
    )
    def segsum(xs, gidx_hbm, dst_hbm, zblk, sums_hbm, *scr):
        gs = scr[:_IBUF]
        ds_ = scr[_IBUF:2 * _IBUF]
        rs = scr[2 * _IBUF:2 * _IBUF + _NBUF]
        acc_sh = scr[2 * _IBUF + _NBUF]
        gsem = scr[2 * _IBUF + _NBUF + 1:2 * _IBUF + 2 * _NBUF + 1]
        isem = scr[2 * _IBUF + 2 * _NBUF + 1:]
        c = lax.axis_index("c")
        s = lax.axis_index("s")

        def fire_idx(j, b):
            base = (j * 16 + s) * _CH
            pltpu.async_copy(gidx_hbm.at[pl.ds(c * E_EDGE + base, _CH)],
                             gs[b], isem[b])
            pltpu.async_copy(dst_hbm.at[pl.ds(base, _CH)], ds_[b], isem[b])

        def drain_idx(b):
            pltpu.make_async_copy(gidx_hbm.at[pl.ds(0, _CH)], gs[b],
                                  isem[b]).wait()
            pltpu.make_async_copy(dst_hbm.at[pl.ds(0, _CH)], ds_[b],
                                  isem[b]).wait()

        GLEAD = _NBUF - 1
        ILEAD = _IBUF - 1

        for b in range(ILEAD):
            fire_idx(b, b)
        for j in range(8):
            ch = s + j * 16

            @pl.when(ch < _NROWCH)
            def _():
                pltpu.sync_copy(zblk, acc_sh.at[pl.ds(ch * _ZROWS, _ZROWS)])
        for b in range(GLEAD):
            drain_idx(b)
            pltpu.async_copy(xs.at[gs[b]], rs[b], gsem[b])
        plsc.subcore_barrier()

        def grp(gi, carry):
            for b8 in range(_IBUF):
                j = gi * _IBUF + b8
                rb = b8 % _NBUF

                @pl.when(j < _NCP)
                def _():
                    pltpu.make_async_copy(xs.at[gs[b8]], rs[rb],
                                          gsem[rb]).wait()
                    pltpu.sync_copy(rs[rb], acc_sh.at[ds_[b8]], add=True)
                    nj8 = j + ILEAD

                    @pl.when(nj8 < _NCP)
                    def _():
                        fire_idx(nj8, (b8 + ILEAD) % _IBUF)
                    nj = j + GLEAD

                    @pl.when(nj < _NCP)
                    def _():
                        ib = (b8 + GLEAD) % _IBUF
                        drain_idx(ib)
                        pltpu.async_copy(xs.at[gs[ib]], rs[(rb + GLEAD) % _NBUF],
                                        gsem[(rb + GLEAD) % _NBUF])
            return carry

        lax.fori_loop(0, (_NCP + _IBUF - 1) // _IBUF, grp, 0)
        plsc.subcore_barrier()
        for j in range(8):
            ch = s + j * 16

            @pl.when(ch < _NROWCH)
            def _():
                r0 = ch * _ZROWS
                pltpu.sync_copy(acc_sh.at[pl.ds(r0, _ZROWS)],
                                sums_hbm.at[c, pl.ds(r0, _ZROWS)])

    return segsum


_CW = HALF


@functools.lru_cache(maxsize=None)
def _make_counts():
    @functools.partial(
        pl.kernel,
        out_type=[jax.ShapeDtypeStruct((2, N_NODE, _CW), f32)],
        mesh=_get_mesh(),
        scratch_types=(
            [pltpu.VMEM((_CH,), jnp.int32)] * _IBUF
            + [
                pltpu.VMEM((_CH, _CW), f32),
                pltpu.VMEM_SHARED((N_NODE, _CW), f32),
            ]
            + [pltpu.SemaphoreType.DMA] * _IBUF
        ),
    )
    def counts(dst2_hbm, zblk, oneblk, cnts_hbm, *scr):
        ds_ = scr[:_IBUF]
        ones_v = scr[_IBUF]
        cnt_sh = scr[_IBUF + 1]
        isem = scr[_IBUF + 2:]
        c = lax.axis_index("c")
        s = lax.axis_index("s")

        def fire_idx(j, b):
            base = c * E_EDGE + (j * 16 + s) * _CH
            pltpu.async_copy(dst2_hbm.at[pl.ds(base, _CH)], ds_[b], isem[b])

        def drain_idx(b):
            pltpu.make_async_copy(dst2_hbm.at[pl.ds(0, _CH)], ds_[b],
                                  isem[b]).wait()

        ILEAD = _IBUF - 1
        for b in range(ILEAD):
            fire_idx(b, b)
        pltpu.sync_copy(oneblk, ones_v)
        for j in range(8):
            ch = s + j * 16

            @pl.when(ch < _NROWCH)
            def _():
                pltpu.sync_copy(zblk, cnt_sh.at[pl.ds(ch * _ZROWS, _ZROWS)])
        plsc.subcore_barrier()

        def grp(gi, carry):
            for b8 in range(_IBUF):
                j = gi * _IBUF + b8

                @pl.when(j < _NCP)
                def _():
                    drain_idx(b8)
                    pltpu.sync_copy(ones_v, cnt_sh.at[ds_[b8]], add=True)
                    nj = j + ILEAD

                    @pl.when(nj < _NCP)
                    def _():
                        fire_idx(nj, (b8 + ILEAD) % _IBUF)
            return carry

        lax.fori_loop(0, (_NCP + _IBUF - 1) // _IBUF, grp, 0)
        plsc.subcore_barrier()
        for j in range(8):
            ch = s + j * 16

            @pl.when(ch < _NROWCH)
            def _():
                r0 = ch * _ZROWS
                pltpu.sync_copy(cnt_sh.at[pl.ds(r0, _ZROWS)],
                                cnts_hbm.at[c, pl.ds(r0, _ZROWS)])

    return counts


_GCH = 80
_NGCH = L_LAB // _GCH


@functools.lru_cache(maxsize=None)
def _make_edge_gather():
    @functools.partial(
        pl.kernel,
        out_type=[jax.ShapeDtypeStruct((L_LAB, D), f32),
                  jax.ShapeDtypeStruct((L_LAB, D), f32)],
        mesh=_get_mesh(),
        scratch_types=[
            pltpu.VMEM((_GCH,), jnp.int32),
            pltpu.VMEM((_GCH,), jnp.int32),
            pltpu.VMEM((_GCH, D), f32),
            pltpu.VMEM((_GCH, D), f32),
            pltpu.SemaphoreType.DMA,
        ],
    )
    def edge_gather(p1, m1, row_hbm, col_hbm, pg_hbm, mg_hbm,
                    idx_v, idx2_v, rows_v, rows2_v, sem):
        c = lax.axis_index("c")
        s = lax.axis_index("s")
        wid = s * 2 + c

        def step(i, carry):
            ch = wid + i * 32

            @pl.when(ch < _NGCH)
            def _():
                base = ch * _GCH
                pltpu.sync_copy(row_hbm.at[pl.ds(base, _GCH)], idx_v)
                pltpu.async_copy(p1.at[idx_v], rows_v, sem).wait()
                pltpu.sync_copy(rows_v, pg_hbm.at[pl.ds(base, _GCH)])
                pltpu.sync_copy(col_hbm.at[pl.ds(base, _GCH)], idx2_v)
                pltpu.async_copy(m1.at[idx2_v], rows2_v, sem).wait()
                pltpu.sync_copy(rows2_v, mg_hbm.at[pl.ds(base, _GCH)])
            return carry

        lax.fori_loop(0, (_NGCH + 31) // 32, step, 0)

    return edge_gather


_BM = 1000


def _make_fused(relu: bool, with_post: bool):
    def body(*refs):
        if with_post:
            (sa_ref, sb_ref, cnt_ref, x_ref, wl_ref, wr_ref, b_ref, wp_ref,
             out_ref) = refs
        else:
            (sa_ref, sb_ref, cnt_ref, x_ref, wl_ref, wr_ref, b_ref,
             out_ref) = refs
        inv = 1.0 / jnp.maximum(cnt_ref[...], 1.0)
        wl = wl_ref[...]
        wr = wr_ref[...]
        b = b_ref[...]
        if with_post:
            wp = wp_ref[...]
            wl = jnp.dot(wl, wp, preferred_element_type=f32)
            wr = jnp.dot(wr, wp, preferred_element_type=f32)
            b = jnp.dot(b, wp, preferred_element_type=f32)
        acc = (jnp.dot(sa_ref[0] * inv, wl[:HALF, :],
                       preferred_element_type=f32)
               + jnp.dot(sb_ref[0] * inv, wl[HALF:, :],
                         preferred_element_type=f32)
               + jnp.dot(x_ref[...], wr, preferred_element_type=f32) + b)
        if relu:
            acc = jnp.maximum(acc, 0.0)
        out_ref[...] = acc

    n_rows = N_NODE // _BM
    in_specs = [
        pl.BlockSpec((1, _BM, HALF), lambda i: (0, i, 0)),
        pl.BlockSpec((1, _BM, HALF), lambda i: (1, i, 0)),
        pl.BlockSpec((_BM, 1), lambda i: (i, 0)),
        pl.BlockSpec((_BM, D), lambda i: (i, 0)),
        pl.BlockSpec((D, D), lambda i: (0, 0)),
        pl.BlockSpec((D, D), lambda i: (0, 0)),
        pl.BlockSpec((1, D), lambda i: (0, 0)),
    ]
    if with_post:
        in_specs.append(pl.BlockSpec((D, D), lambda i: (0, 0)))
    return pl.pallas_call(
        body,
        grid=(n_rows,),
        in_specs=in_specs,
        out_specs=pl.BlockSpec((_BM, D), lambda i: (i, 0)),
        out_shape=jax.ShapeDtypeStruct((N_NODE, D), f32),
    )


_fused_relu = _make_fused(True, False)
_fused_post = _make_fused(False, True)

_BL = 2000


def _dec_body(pg_ref, mg_ref, bd1_ref, wd2_ref, bd2_ref, out_ref):
    z = jnp.maximum(pg_ref[...] + mg_ref[...] + bd1_ref[...], 0.0)
    out_ref[...] = jnp.dot(z, wd2_ref[...], preferred_element_type=f32) + bd2_ref[...]


_decoder = pl.pallas_call(
    _dec_body,
    grid=(L_LAB // _BL,),
    in_specs=[
        pl.BlockSpec((_BL, D), lambda i: (i, 0)),
        pl.BlockSpec((_BL, D), lambda i: (i, 0)),
        pl.BlockSpec((1, D), lambda i: (0, 0)),
        pl.BlockSpec((D, 1), lambda i: (0, 0)),
        pl.BlockSpec((1, 1), lambda i: (0, 0)),
    ],
    out_specs=pl.BlockSpec((_BL, 1), lambda i: (i, 0)),
    out_shape=jax.ShapeDtypeStruct((L_LAB, 1), f32),
)


def kernel(x_patient, x_medicine, edge_index_pm, edge_index_mp,
           edge_label_index, W1l_pm, W1r_pm, b1_pm, W1l_mp, W1r_mp, b1_mp,
           W2l_pm, W2r_pm, b2_pm, W2l_mp, W2r_mp, b2_mp, Wd1, bd1, Wd2, bd2):
    i32 = jnp.int32
    src_pm = edge_index_pm[0].astype(i32)
    dst_pm = edge_index_pm[1].astype(i32)
    src_mp = edge_index_mp[0].astype(i32)
    dst_mp = edge_index_mp[1].astype(i32)
    row = edge_label_index[0].astype(i32)
    col = edge_label_index[1].astype(i32)

    zblk = jnp.zeros((_ZROWS, HALF), f32)
    zblk_c = jnp.zeros((_ZROWS, _CW), f32)
    oneblk = jnp.ones((_CH, _CW), f32)

    _segsum = _make_segsum()
    _counts = _make_counts()
    _edge_gather = _make_edge_gather()

    dst2 = jnp.concatenate([dst_pm, dst_mp])
    (Cboth,) = _counts(dst2, zblk_c, oneblk)
    cnt_pm = Cboth[0, :, 0:1]
    cnt_mp = Cboth[1, :, 0:1]

    gidx_pm = jnp.concatenate([src_pm * 2, src_pm * 2 + 1])
    gidx_mp = jnp.concatenate([src_mp * 2, src_mp * 2 + 1])

    (S1pm,) = _segsum(x_patient.reshape(2 * N_NODE, HALF), gidx_pm, dst_pm,
                      zblk)
    (S1mp,) = _segsum(x_medicine.reshape(2 * N_NODE, HALF), gidx_mp, dst_mp,
                      zblk)

    h_m = _fused_relu(S1pm, S1pm, cnt_pm, x_medicine, W1l_pm, W1r_pm,
                      b1_pm.reshape(1, D))
    (S2mp,) = _segsum(h_m.reshape(2 * N_NODE, HALF), gidx_mp, dst_mp, zblk)
    h_p = _fused_relu(S1mp, S1mp, cnt_mp, x_patient, W1l_mp, W1r_mp,
                      b1_mp.reshape(1, D))
    (S2pm,) = _segsum(h_p.reshape(2 * N_NODE, HALF), gidx_pm, dst_pm, zblk)

    P1 = _fused_post(S2mp, S2mp, cnt_mp, h_p, W2l_mp, W2r_mp,
                     b2_mp.reshape(1, D), Wd1[:D, :])
    M1 = _fused_post(S2pm, S2pm, cnt_pm, h_m, W2l_pm, W2r_pm,
                     b2_pm.reshape(1, D), Wd1[D:, :])

    Pg, Mg = _edge_gather(P1, M1, row, col)
    return _decoder(Pg, Mg, bd1.reshape(1, D), Wd2, bd2.reshape(1, 1))


# speedup vs baseline: 6.0297x; 1.0107x over previous
import functools

import jax
import jax.numpy as jnp
from jax import lax
from jax.experimental import pallas as pl
from jax.experimental.pallas import tpu as pltpu
from jax.experimental.pallas import tpu_sc as plsc

N_NODE = 10000
E_EDGE = 160000
L_LAB = 20000
D = 256
HALF = 128

_CH = 80
_NCHUNK = E_EDGE // _CH
_ZROWS = 80
_NROWCH = N_NODE // _ZROWS

f32 = jnp.float32


@functools.lru_cache(maxsize=None)
def _get_mesh():
    return plsc.VectorSubcoreMesh(core_axis_name="c", subcore_axis_name="s")


_NBUF = 4
_IBUF = 2 * _NBUF
_NCP = _NCHUNK // 16


@functools.lru_cache(maxsize=None)
def _make_segsum():
    @functools.partial(
        pl.kernel,
        out_type=[jax.ShapeDtypeStruct((2, N_NODE, HALF), f32)],
        mesh=_get_mesh(),
        scratch_types=(
            [pltpu.VMEM((_CH,), jnp.int32)] * _IBUF
            + [pltpu.VMEM((_CH,), jnp.int32)] * _IBUF
            + [pltpu.VMEM((_CH, HALF), f32)] * _NBUF
            + [
                pltpu.VMEM_SHARED((N_NODE, HALF), f32),
            ]
            + [pltpu.SemaphoreType.DMA] * _NBUF
            + [pltpu.SemaphoreType.DMA] * _IBUF
        ),
    )
    def segsum(xs, gidx_hbm, dst_hbm, zblk, sums_hbm, *scr):
        gs = scr[:_IBUF]
        ds_ = scr[_IBUF:2 * _IBUF]
        rs = scr[2 * _IBUF:2 * _IBUF + _NBUF]
        acc_sh = scr[2 * _IBUF + _NBUF]
        gsem = scr[2 * _IBUF + _NBUF + 1:2 * _IBUF + 2 * _NBUF + 1]
        isem = scr[2 * _IBUF + 2 * _NBUF + 1:]
        c = lax.axis_index("c")
        s = lax.axis_index("s")

        def fire_idx(j, b):
            base = (j * 16 + s) * _CH
            pltpu.async_copy(gidx_hbm.at[pl.ds(c * E_EDGE + base, _CH)],
                             gs[b], isem[b])
            pltpu.async_copy(dst_hbm.at[pl.ds(base, _CH)], ds_[b], isem[b])

        def drain_idx(b):
            pltpu.make_async_copy(gidx_hbm.at[pl.ds(0, _CH)], gs[b],
                                  isem[b]).wait()
            pltpu.make_async_copy(dst_hbm.at[pl.ds(0, _CH)], ds_[b],
                                  isem[b]).wait()

        GLEAD = _NBUF - 1
        ILEAD = _IBUF - 1

        for b in range(ILEAD):
            fire_idx(b, b)
        for j in range(8):
            ch = s + j * 16

            @pl.when(ch < _NROWCH)
            def _():
                pltpu.sync_copy(zblk, acc_sh.at[pl.ds(ch * _ZROWS, _ZROWS)])
        for b in range(GLEAD):
            drain_idx(b)
            pltpu.async_copy(xs.at[gs[b]], rs[b], gsem[b])
        plsc.subcore_barrier()

        def grp(gi, carry):
            for b8 in range(_IBUF):
                j = gi * _IBUF + b8
                rb = b8 % _NBUF

                @pl.when(j < _NCP)
                def _():
                    pltpu.make_async_copy(xs.at[gs[b8]], rs[rb],
                                          gsem[rb]).wait()
                    pltpu.sync_copy(rs[rb], acc_sh.at[ds_[b8]], add=True)
                    nj8 = j + ILEAD

                    @pl.when(nj8 < _NCP)
                    def _():
                        fire_idx(nj8, (b8 + ILEAD) % _IBUF)
                    nj = j + GLEAD

                    @pl.when(nj < _NCP)
                    def _():
                        ib = (b8 + GLEAD) % _IBUF
                        drain_idx(ib)
                        pltpu.async_copy(xs.at[gs[ib]], rs[(rb + GLEAD) % _NBUF],
                                        gsem[(rb + GLEAD) % _NBUF])
            return carry

        lax.fori_loop(0, (_NCP + _IBUF - 1) // _IBUF, grp, 0)
        plsc.subcore_barrier()
        for j in range(8):
            ch = s + j * 16

            @pl.when(ch < _NROWCH)
            def _():
                r0 = ch * _ZROWS
                pltpu.sync_copy(acc_sh.at[pl.ds(r0, _ZROWS)],
                                sums_hbm.at[c, pl.ds(r0, _ZROWS)])

    return segsum


_CW = HALF


@functools.lru_cache(maxsize=None)
def _make_counts():
    @functools.partial(
        pl.kernel,
        out_type=[jax.ShapeDtypeStruct((2, N_NODE, _CW), f32)],
        mesh=_get_mesh(),
        scratch_types=(
            [pltpu.VMEM((_CH,), jnp.int32)] * _IBUF
            + [
                pltpu.VMEM((_CH, _CW), f32),
                pltpu.VMEM_SHARED((N_NODE, _CW), f32),
            ]
            + [pltpu.SemaphoreType.DMA] * _IBUF
        ),
    )
    def counts(dst2_hbm, zblk, oneblk, cnts_hbm, *scr):
        ds_ = scr[:_IBUF]
        ones_v = scr[_IBUF]
        cnt_sh = scr[_IBUF + 1]
        isem = scr[_IBUF + 2:]
        c = lax.axis_index("c")
        s = lax.axis_index("s")

        def fire_idx(j, b):
            base = c * E_EDGE + (j * 16 + s) * _CH
            pltpu.async_copy(dst2_hbm.at[pl.ds(base, _CH)], ds_[b], isem[b])

        def drain_idx(b):
            pltpu.make_async_copy(dst2_hbm.at[pl.ds(0, _CH)], ds_[b],
                                  isem[b]).wait()

        ILEAD = _IBUF - 1
        for b in range(ILEAD):
            fire_idx(b, b)
        pltpu.sync_copy(oneblk, ones_v)
        for j in range(8):
            ch = s + j * 16

            @pl.when(ch < _NROWCH)
            def _():
                pltpu.sync_copy(zblk, cnt_sh.at[pl.ds(ch * _ZROWS, _ZROWS)])
        plsc.subcore_barrier()

        def grp(gi, carry):
            for b8 in range(_IBUF):
                j = gi * _IBUF + b8

                @pl.when(j < _NCP)
                def _():
                    drain_idx(b8)
                    pltpu.sync_copy(ones_v, cnt_sh.at[ds_[b8]], add=True)
                    nj = j + ILEAD

                    @pl.when(nj < _NCP)
                    def _():
                        fire_idx(nj, (b8 + ILEAD) % _IBUF)
            return carry

        lax.fori_loop(0, (_NCP + _IBUF - 1) // _IBUF, grp, 0)
        plsc.subcore_barrier()
        for j in range(8):
            ch = s + j * 16

            @pl.when(ch < _NROWCH)
            def _():
                r0 = ch * _ZROWS
                pltpu.sync_copy(cnt_sh.at[pl.ds(r0, _ZROWS)],
                                cnts_hbm.at[c, pl.ds(r0, _ZROWS)])

    return counts


_GCH = 80
_NGCH = L_LAB // _GCH


@functools.lru_cache(maxsize=None)
def _make_edge_gather():
    @functools.partial(
        pl.kernel,
        out_type=[jax.ShapeDtypeStruct((L_LAB, D), f32),
                  jax.ShapeDtypeStruct((L_LAB, D), f32)],
        mesh=_get_mesh(),
        scratch_types=(
            [pltpu.VMEM((_GCH,), jnp.int32)] * 4
            + [pltpu.VMEM((_GCH, D), f32)] * 4
            + [pltpu.SemaphoreType.DMA] * 12
        ),
    )
    def edge_gather(p1, m1, row_hbm, col_hbm, pg_hbm, mg_hbm, *scr):
        idxP = scr[0:2]
        idxM = scr[2:4]
        rowsP = scr[4:6]
        rowsM = scr[6:8]
        isemP = scr[8:10]
        isemM = scr[10:12]
        gsemP = scr[12:14]
        gsemM = scr[14:16]
        wsemP = scr[16:18]
        wsemM = scr[18:20]
        c = lax.axis_index("c")
        s = lax.axis_index("s")
        wid = s * 2 + c

        def fire_idx(i, b):
            base = (wid + i * 32) * _GCH
            pltpu.async_copy(row_hbm.at[pl.ds(base, _GCH)], idxP[b], isemP[b])
            pltpu.async_copy(col_hbm.at[pl.ds(base, _GCH)], idxM[b], isemM[b])

        def drain_wb(b):
            pltpu.make_async_copy(rowsP[b], pg_hbm.at[pl.ds(0, _GCH)],
                                  wsemP[b]).wait()
            pltpu.make_async_copy(rowsM[b], mg_hbm.at[pl.ds(0, _GCH)],
                                  wsemM[b]).wait()

        fire_idx(0, 0)

        def grp(g, carry):
            for b in range(2):
                i = g * 2 + b
                ch = wid + i * 32

                @pl.when(ch < _NGCH)
                def _():
                    @pl.when(i >= 2)
                    def _():
                        drain_wb(b)

                    @pl.when(wid + (i + 1) * 32 < _NGCH)
                    def _():
                        fire_idx(i + 1, 1 - b)
                    pltpu.make_async_copy(row_hbm.at[pl.ds(0, _GCH)],
                                          idxP[b], isemP[b]).wait()
                    pltpu.async_copy(p1.at[idxP[b]], rowsP[b], gsemP[b])
                    pltpu.make_async_copy(col_hbm.at[pl.ds(0, _GCH)],
                                          idxM[b], isemM[b]).wait()
                    pltpu.async_copy(m1.at[idxM[b]], rowsM[b], gsemM[b])
                    base = ch * _GCH
                    pltpu.make_async_copy(p1.at[idxP[b]], rowsP[b],
                                          gsemP[b]).wait()
                    pltpu.async_copy(rowsP[b], pg_hbm.at[pl.ds(base, _GCH)],
                                     wsemP[b])
                    pltpu.make_async_copy(m1.at[idxM[b]], rowsM[b],
                                          gsemM[b]).wait()
                    pltpu.async_copy(rowsM[b], mg_hbm.at[pl.ds(base, _GCH)],
                                     wsemM[b])
            return carry

        nit = (_NGCH + 31) // 32
        lax.fori_loop(0, nit // 2, grp, 0)
        drain_wb(0)

        @pl.when(wid + (nit - 1) * 32 < _NGCH)
        def _():
            drain_wb(1)

    return edge_gather


_BM = 1000


def _make_fused(relu: bool, with_post: bool):
    def body(*refs):
        if with_post:
            (sa_ref, sb_ref, cnt_ref, x_ref, wl_ref, wr_ref, b_ref, wp_ref,
             out_ref) = refs
        else:
            (sa_ref, sb_ref, cnt_ref, x_ref, wl_ref, wr_ref, b_ref,
             out_ref) = refs
        inv = 1.0 / jnp.maximum(cnt_ref[...], 1.0)
        wl = wl_ref[...]
        wr = wr_ref[...]
        b = b_ref[...]
        if with_post:
            wp = wp_ref[...]
            wl = jnp.dot(wl, wp, preferred_element_type=f32)
            wr = jnp.dot(wr, wp, preferred_element_type=f32)
            b = jnp.dot(b, wp, preferred_element_type=f32)
        acc = (jnp.dot(sa_ref[0] * inv, wl[:HALF, :],
                       preferred_element_type=f32)
               + jnp.dot(sb_ref[0] * inv, wl[HALF:, :],
                         preferred_element_type=f32)
               + jnp.dot(x_ref[...], wr, preferred_element_type=f32) + b)
        if relu:
            acc = jnp.maximum(acc, 0.0)
        out_ref[...] = acc

    n_rows = N_NODE // _BM
    in_specs = [
        pl.BlockSpec((1, _BM, HALF), lambda i: (0, i, 0)),
        pl.BlockSpec((1, _BM, HALF), lambda i: (1, i, 0)),
        pl.BlockSpec((_BM, 1), lambda i: (i, 0)),
        pl.BlockSpec((_BM, D), lambda i: (i, 0)),
        pl.BlockSpec((D, D), lambda i: (0, 0)),
        pl.BlockSpec((D, D), lambda i: (0, 0)),
        pl.BlockSpec((1, D), lambda i: (0, 0)),
    ]
    if with_post:
        in_specs.append(pl.BlockSpec((D, D), lambda i: (0, 0)))
    return pl.pallas_call(
        body,
        grid=(n_rows,),
        in_specs=in_specs,
        out_specs=pl.BlockSpec((_BM, D), lambda i: (i, 0)),
        out_shape=jax.ShapeDtypeStruct((N_NODE, D), f32),
    )


_fused_relu = _make_fused(True, False)
_fused_post = _make_fused(False, True)

_BL = 2000


def _dec_body(pg_ref, mg_ref, bd1_ref, wd2_ref, bd2_ref, out_ref):
    z = jnp.maximum(pg_ref[...] + mg_ref[...] + bd1_ref[...], 0.0)
    out_ref[...] = jnp.dot(z, wd2_ref[...], preferred_element_type=f32) + bd2_ref[...]


_decoder = pl.pallas_call(
    _dec_body,
    grid=(L_LAB // _BL,),
    in_specs=[
        pl.BlockSpec((_BL, D), lambda i: (i, 0)),
        pl.BlockSpec((_BL, D), lambda i: (i, 0)),
        pl.BlockSpec((1, D), lambda i: (0, 0)),
        pl.BlockSpec((D, 1), lambda i: (0, 0)),
        pl.BlockSpec((1, 1), lambda i: (0, 0)),
    ],
    out_specs=pl.BlockSpec((_BL, 1), lambda i: (i, 0)),
    out_shape=jax.ShapeDtypeStruct((L_LAB, 1), f32),
)


def kernel(x_patient, x_medicine, edge_index_pm, edge_index_mp,
           edge_label_index, W1l_pm, W1r_pm, b1_pm, W1l_mp, W1r_mp, b1_mp,
           W2l_pm, W2r_pm, b2_pm, W2l_mp, W2r_mp, b2_mp, Wd1, bd1, Wd2, bd2):
    i32 = jnp.int32
    src_pm = edge_index_pm[0].astype(i32)
    dst_pm = edge_index_pm[1].astype(i32)
    src_mp = edge_index_mp[0].astype(i32)
    dst_mp = edge_index_mp[1].astype(i32)
    row = edge_label_index[0].astype(i32)
    col = edge_label_index[1].astype(i32)

    zblk = jnp.zeros((_ZROWS, HALF), f32)
    zblk_c = jnp.zeros((_ZROWS, _CW), f32)
    oneblk = jnp.ones((_CH, _CW), f32)

    _segsum = _make_segsum()
    _counts = _make_counts()
    _edge_gather = _make_edge_gather()

    dst2 = jnp.concatenate([dst_pm, dst_mp])
    (Cboth,) = _counts(dst2, zblk_c, oneblk)
    cnt_pm = Cboth[0, :, 0:1]
    cnt_mp = Cboth[1, :, 0:1]

    gidx_pm = jnp.concatenate([src_pm * 2, src_pm * 2 + 1])
    gidx_mp = jnp.concatenate([src_mp * 2, src_mp * 2 + 1])

    (S1pm,) = _segsum(x_patient.reshape(2 * N_NODE, HALF), gidx_pm, dst_pm,
                      zblk)
    (S1mp,) = _segsum(x_medicine.reshape(2 * N_NODE, HALF), gidx_mp, dst_mp,
                      zblk)

    h_m = _fused_relu(S1pm, S1pm, cnt_pm, x_medicine, W1l_pm, W1r_pm,
                      b1_pm.reshape(1, D))
    (S2mp,) = _segsum(h_m.reshape(2 * N_NODE, HALF), gidx_mp, dst_mp, zblk)
    h_p = _fused_relu(S1mp, S1mp, cnt_mp, x_patient, W1l_mp, W1r_mp,
                      b1_mp.reshape(1, D))
    (S2pm,) = _segsum(h_p.reshape(2 * N_NODE, HALF), gidx_pm, dst_pm, zblk)

    P1 = _fused_post(S2mp, S2mp, cnt_mp, h_p, W2l_mp, W2r_mp,
                     b2_mp.reshape(1, D), Wd1[:D, :])
    M1 = _fused_post(S2pm, S2pm, cnt_pm, h_m, W2l_pm, W2r_pm,
                     b2_pm.reshape(1, D), Wd1[D:, :])

    Pg, Mg = _edge_gather(P1, M1, row, col)
    return _decoder(Pg, Mg, bd1.reshape(1, D), Wd2, bd2.reshape(1, 1))
